# baseline (speedup 1.0000x reference)
import functools

import jax
import jax.numpy as jnp
from jax import lax
from jax.experimental import pallas as pl
from jax.experimental.pallas import tpu as pltpu
from jax.experimental.pallas import tpu_sc as plsc

_RESOLUTIONS = (17, 33, 65, 129)
_NUM_POINTS = (0, 8000, 8000, 8000)
_FINAL = 129
_BLK = 512
_K = 8000
_KPAD = 8192
_NSUB = 16
_CH = 2048


def _mlp_body(pts_ref, feats_ref, w1p_ref, w1f_ref, b1_ref, w2_ref, b2_ref,
              w3_ref, b3_ref, out_ref):
    c1 = jnp.dot(feats_ref[...], w1f_ref[...],
                 preferred_element_type=jnp.float32) + b1_ref[...]
    h = jnp.dot(pts_ref[...], w1p_ref[...],
                preferred_element_type=jnp.float32) + c1
    h = jnp.maximum(h, 0.0)
    h = jnp.dot(h, w2_ref[...], preferred_element_type=jnp.float32) + b2_ref[...]
    h = jnp.maximum(h, 0.0)
    o = jnp.dot(h, w3_ref[...], preferred_element_type=jnp.float32) + b3_ref[...]
    out_ref[...] = jax.nn.sigmoid(o)


def _mlp_eval(pts_pad, feats, w1p, w1f, b1, w2, b2, w3p, b3p):
    n_pad = pts_pad.shape[0]
    grid = (n_pad // _BLK,)
    full = lambda shape: pl.BlockSpec(shape, lambda i: (0, 0))
    return pl.pallas_call(
        _mlp_body,
        grid=grid,
        in_specs=[
            pl.BlockSpec((_BLK, 128), lambda i: (i, 0)),
            full((1, 256)), full((128, 256)), full((256, 256)),
            full((1, 256)), full((256, 256)), full((1, 256)),
            full((256, 128)), full((1, 128)),
        ],
        out_specs=pl.BlockSpec((_BLK, 128), lambda i: (i, 0)),
        out_shape=jax.ShapeDtypeStruct((n_pad, 128), jnp.float32),
    )(pts_pad, feats, w1p, w1f, b1, w2, b2, w3p, b3p)


def _pad_points(coords3d):
    n = coords3d.shape[0]
    n_pad = (n + _BLK - 1) // _BLK * _BLK
    pts = jnp.zeros((n_pad, 128), jnp.float32)
    return pts.at[:n, :3].set(coords3d)


def _scale(coords):
    c = coords.astype(jnp.float32) / (_FINAL - 1)
    return c * 2.0 - 1.0


def _upsample(vol):
    def up_last(v):
        d = v.shape[-1]
        mid = 0.5 * v[..., :-1] + 0.5 * v[..., 1:]
        stacked = jnp.stack([v[..., :-1], mid], axis=-1).reshape(v.shape[:-1] + (2 * (d - 1),))
        return jnp.concatenate([stacked, v[..., -1:]], axis=-1)
    v = up_last(vol)
    v = up_last(v.transpose(0, 2, 1)).transpose(0, 2, 1)
    v = up_last(v.transpose(2, 1, 0)).transpose(2, 1, 0)
    return v


def _make_select(n, k):
    s_w = -(-n // (_NSUB * _CH)) * _CH
    n_pad = s_w * _NSUB
    n_ch = s_w // _CH
    mesh = plsc.VectorSubcoreMesh(core_axis_name="c", subcore_axis_name="s",
                                  num_cores=1)

    def body(occ_hbm, sel_hbm, stage, hist16, histl, ghist, cnt16, cntall,
             vals2d, pos2d, sem):
        w = lax.axis_index("s")
        base_w = w * s_w
        lanes = lax.iota(jnp.int32, 16)
        ones = jnp.ones((16,), jnp.int32)

        def scan(on_vec, carry):
            def chunk(c, carry):
                pltpu.sync_copy(occ_hbm.at[pl.ds(base_w + c * _CH, _CH)], stage)
                def vec(i, carry):
                    b = stage[pl.ds(i * 16, 16)]
                    g = base_w + c * _CH + i * 16 + lanes
                    return on_vec(b, g, carry)
                return lax.fori_loop(0, _CH // 16, vec, carry)
            return lax.fori_loop(0, n_ch, chunk, carry)

        prefix = jnp.int32(0)
        k_rem = jnp.int32(k)
        total_lt = jnp.int32(0)
        local_lt = jnp.int32(0)
        n_eq_w = jnp.int32(0)

        for p in range(4):
            shift = 24 - 8 * p

            def z16(i, _):
                hist16[pl.ds(i * 16, 16)] = jnp.zeros((16,), jnp.int32)
                return None
            lax.fori_loop(0, 256, z16, None)

            def hvec(b, g, carry, shift=shift, prefix=prefix, p=p):
                slot = ((b >> shift) & 0xFF) * 16 + lanes
                if p == 0:
                    plsc.addupdate_scatter(hist16, [slot], ones)
                else:
                    m = (b >> (shift + 8)) == prefix
                    plsc.addupdate_scatter(hist16, [slot], ones, mask=m)
                return carry
            scan(hvec, None)

            def red(gq, _):
                def addl(l, acc):
                    return acc + hist16[pl.ds(l * 256 + gq * 16, 16)]
                histl[pl.ds(gq * 16, 16)] = lax.fori_loop(
                    0, 16, addl, jnp.zeros((16,), jnp.int32))
                return None
            lax.fori_loop(0, 16, red, None)

            pltpu.sync_copy(histl, cntall.at[pl.ds(w * 256, 256)])
            plsc.subcore_barrier()
            pltpu.sync_copy(cntall, hist16)
            plsc.subcore_barrier()
            def gred(gq, _):
                def addu(u, acc):
                    return acc + hist16[pl.ds(u * 256 + gq * 16, 16)]
                ghist[pl.ds(gq * 16, 16)] = lax.fori_loop(
                    0, 16, addu, jnp.zeros((16,), jnp.int32))
                return None
            lax.fori_loop(0, 16, gred, None)

            def find(gq, carry):
                running, bucket, c_less = carry
                v = ghist[pl.ds(gq * 16, 16)]
                cum = plsc.cumsum(v)
                tot = jnp.max(cum)
                ffs = jnp.max(plsc.all_reduce_ffs((running + cum) >= k_rem))
                less = jnp.sum(jnp.where(lanes < ffs, v, 0))
                new = jnp.logical_and(bucket < 0, ffs < 16)
                bucket = jnp.where(new, gq * 16 + ffs, bucket)
                c_less = jnp.where(new, running + less, c_less)
                return running + tot, bucket, c_less
            _, bucket, c_less = lax.fori_loop(
                0, 16, find, (jnp.int32(0), jnp.int32(-1), jnp.int32(0)))

            def lcnt(gq, carry):
                ll, le = carry
                v = histl[pl.ds(gq * 16, 16)]
                slot = gq * 16 + lanes
                ll = ll + jnp.sum(jnp.where(slot < bucket, v, 0))
                le = le + jnp.sum(jnp.where(slot == bucket, v, 0))
                return ll, le
            ll, le = lax.fori_loop(0, 16, lcnt, (jnp.int32(0), jnp.int32(0)))
            local_lt = local_lt + ll
            n_eq_w = le
            total_lt = total_lt + c_less
            k_rem = k_rem - c_less
            prefix = (prefix << 8) | bucket

        thresh = prefix

        cnt16[...] = jnp.where(lanes == 0, local_lt,
                               jnp.where(lanes == 1, n_eq_w, 0))
        pltpu.sync_copy(cnt16, cntall.at[pl.ds(w * 16, 16)])
        plsc.subcore_barrier()
        pltpu.sync_copy(cntall.at[pl.ds(0, 256)], hist16.at[pl.ds(0, 256)])
        lt_off = jnp.int32(0)
        eq_off = jnp.int32(0)
        for u in range(_NSUB):
            take = u < w
            cu = hist16[pl.ds(u * 16, 16)]
            lt_off = lt_off + jnp.where(take, cu[0], 0)
            eq_off = eq_off + jnp.where(take, cu[1], 0)

        def init_pos(i, _):
            pos2d[i >> 3, pl.ds((i & 7) * 16, 16)] = jnp.full(
                (16,), _KPAD - 2, jnp.int32)
            return None
        lax.fori_loop(0, _KPAD // 16, init_pos, None)

        def evec(b, g, carry):
            ltc, eqr, lc = carry
            lt = b < thresh
            eq = b == thresh
            pos_lt = ltc + plsc.cumsum(lt.astype(jnp.int32)) - 1
            reqr = eqr + plsc.cumsum(eq.astype(jnp.int32)) - 1
            acc = jnp.logical_and(eq, reqr < k_rem)
            pos = jnp.where(lt, pos_lt, total_lt + reqr)
            sel = jnp.logical_or(lt, acc)
            slot = lc + plsc.cumsum(sel.astype(jnp.int32)) - 1
            plsc.store_scatter(vals2d, [slot >> 7, slot & 127], g, mask=sel)
            plsc.store_scatter(pos2d, [slot >> 7, slot & 127], pos, mask=sel)
            nlt = jnp.max(plsc.all_reduce_population_count(lt))
            neq = jnp.max(plsc.all_reduce_population_count(eq))
            nsel = jnp.max(plsc.all_reduce_population_count(sel))
            return ltc + nlt, eqr + neq, lc + nsel
        scan(evec, (lt_off, eq_off, jnp.int32(0)))

        def put(j, _):
            pltpu.async_copy(vals2d.at[j], sel_hbm.at[pos2d.at[j]], sem).wait()
            return None
        lax.fori_loop(0, _KPAD // 128, put, None)

    sel_fn = pl.kernel(
        body,
        out_type=jax.ShapeDtypeStruct((_KPAD,), jnp.int32),
        mesh=mesh,
        compiler_params=pltpu.CompilerParams(needs_layout_passes=False),
        scratch_types=[
            pltpu.VMEM((_CH,), jnp.int32),
            pltpu.VMEM((4096,), jnp.int32),
            pltpu.VMEM((256,), jnp.int32),
            pltpu.VMEM((256,), jnp.int32),
            pltpu.VMEM((16,), jnp.int32),
            pltpu.VMEM_SHARED((4096,), jnp.int32),
            pltpu.VMEM((_KPAD // 128, 128), jnp.int32),
            pltpu.VMEM((_KPAD // 128, 128), jnp.int32),
            pltpu.SemaphoreType.DMA,
        ],
    )

    def run(flat):
        bits = lax.bitcast_convert_type(jnp.abs(flat - 0.5), jnp.int32)
        pad = jnp.full((n_pad - n,), 0x7F000000, jnp.int32)
        return sel_fn(jnp.concatenate([bits, pad]))[:k]

    return run


def kernel(feats, W1, b1, W2, b2, W3, b3):
    feats2 = feats.reshape(1, 256)
    w1p = jnp.zeros((128, 256), jnp.float32).at[:3].set(W1[:3])
    w1f = W1[3:]
    b1r = b1.reshape(1, 256)
    b2r = b2.reshape(1, 256)
    w3p = jnp.zeros((256, 128), jnp.float32).at[:, :1].set(W3)
    b3p = jnp.zeros((1, 128), jnp.float32).at[0, 0].set(b3[0])
    mlp = functools.partial(_mlp_eval, feats=feats2, w1p=w1p, w1f=w1f, b1=b1r,
                            w2=W2, b2=b2r, w3p=w3p, b3p=b3p)

    r0 = _RESOLUTIONS[0]
    a = jnp.linspace(0, _FINAL - 1, r0).astype(jnp.int32)
    gz, gy, gx = jnp.meshgrid(a, a, a, indexing='ij')
    coords0 = jnp.stack([gx, gy, gz], axis=0).reshape(3, -1).T
    occ0 = mlp(_pad_points(_scale(coords0)))[:r0 ** 3, 0]
    vol = occ0.reshape(r0, r0, r0)

    for res, num_pt in zip(_RESOLUTIONS[1:], _NUM_POINTS[1:]):
        stride = (_FINAL - 1) // (res - 1)
        vol = _upsample(vol)
        n = res ** 3
        flat = vol.reshape(n)
        idx = _make_select(n, num_pt)(flat)
        xi = idx % res
        yi = (idx // res) % res
        zi = idx // (res * res)
        coords = jnp.stack([xi, yi, zi], axis=-1) * stride
        vals = mlp(_pad_points(_scale(coords)))[:num_pt, 0]
        flat = flat.at[idx].set(vals)
        vol = flat.reshape(res, res, res)

    return vol.reshape(1, 1, _FINAL, _FINAL, _FINAL)


# speedup vs baseline: 43.5542x; 43.5542x over previous
import functools

import jax
import jax.numpy as jnp
from jax import lax
from jax.experimental import pallas as pl
from jax.experimental.pallas import tpu as pltpu
from jax.experimental.pallas import tpu_sc as plsc

_RESOLUTIONS = (17, 33, 65, 129)
_NUM_POINTS = (0, 8000, 8000, 8000)
_FINAL = 129
_BLK = 512
_K = 8000
_KPAD = 8192
_NSUB = 16
_CH = 2048


def _mlp_body(pts_ref, feats_ref, w1p_ref, w1f_ref, b1_ref, w2_ref, b2_ref,
              w3_ref, b3_ref, out_ref):
    c1 = jnp.dot(feats_ref[...], w1f_ref[...],
                 preferred_element_type=jnp.float32) + b1_ref[...]
    h = jnp.dot(pts_ref[...], w1p_ref[...],
                preferred_element_type=jnp.float32) + c1
    h = jnp.maximum(h, 0.0)
    h = jnp.dot(h, w2_ref[...], preferred_element_type=jnp.float32) + b2_ref[...]
    h = jnp.maximum(h, 0.0)
    o = jnp.dot(h, w3_ref[...], preferred_element_type=jnp.float32) + b3_ref[...]
    out_ref[...] = jax.nn.sigmoid(o)


def _mlp_eval(pts_pad, feats, w1p, w1f, b1, w2, b2, w3p, b3p):
    n_pad = pts_pad.shape[0]
    grid = (n_pad // _BLK,)
    full = lambda shape: pl.BlockSpec(shape, lambda i: (0, 0))
    return pl.pallas_call(
        _mlp_body,
        grid=grid,
        in_specs=[
            pl.BlockSpec((_BLK, 128), lambda i: (i, 0)),
            full((1, 256)), full((128, 256)), full((256, 256)),
            full((1, 256)), full((256, 256)), full((1, 256)),
            full((256, 128)), full((1, 128)),
        ],
        out_specs=pl.BlockSpec((_BLK, 128), lambda i: (i, 0)),
        out_shape=jax.ShapeDtypeStruct((n_pad, 128), jnp.float32),
    )(pts_pad, feats, w1p, w1f, b1, w2, b2, w3p, b3p)


def _pad_points(coords3d):
    n = coords3d.shape[0]
    n_pad = (n + _BLK - 1) // _BLK * _BLK
    pts = jnp.zeros((n_pad, 128), jnp.float32)
    return pts.at[:n, :3].set(coords3d)


def _scale(coords):
    c = coords.astype(jnp.float32) / (_FINAL - 1)
    return c * 2.0 - 1.0


def _upsample(vol):
    def up_last(v):
        d = v.shape[-1]
        mid = 0.5 * v[..., :-1] + 0.5 * v[..., 1:]
        stacked = jnp.stack([v[..., :-1], mid], axis=-1).reshape(v.shape[:-1] + (2 * (d - 1),))
        return jnp.concatenate([stacked, v[..., -1:]], axis=-1)
    v = up_last(vol)
    v = up_last(v.transpose(0, 2, 1)).transpose(0, 2, 1)
    v = up_last(v.transpose(2, 1, 0)).transpose(2, 1, 0)
    return v


def _make_select(n, k):
    s_w = -(-n // (_NSUB * _CH)) * _CH
    n_pad = s_w * _NSUB
    n_ch = s_w // _CH
    mesh = plsc.VectorSubcoreMesh(core_axis_name="c", subcore_axis_name="s",
                                  num_cores=1)

    def body(occ_hbm, sel_hbm, stage, hist16, histl, ghist, cnt16, cntall,
             vals2d, pos2d, sem):
        w = lax.axis_index("s")
        base_w = w * s_w
        lanes = lax.iota(jnp.int32, 16)
        ones = jnp.ones((16,), jnp.int32)

        def scan(on_vec, carry):
            def chunk(c, carry):
                pltpu.sync_copy(occ_hbm.at[pl.ds(base_w + c * _CH, _CH)], stage)
                def vec(i, carry):
                    b = stage[pl.ds(i * 16, 16)]
                    g = base_w + c * _CH + i * 16 + lanes
                    return on_vec(b, g, carry)
                return lax.fori_loop(0, _CH // 16, vec, carry)
            return lax.fori_loop(0, n_ch, chunk, carry)

        prefix = jnp.int32(0)
        k_rem = jnp.int32(k)
        total_lt = jnp.int32(0)
        local_lt = jnp.int32(0)
        n_eq_w = jnp.int32(0)

        for p in range(4):
            shift = 24 - 8 * p

            def z16(i, _):
                hist16[pl.ds(i * 16, 16)] = jnp.zeros((16,), jnp.int32)
                return None
            lax.fori_loop(0, 256, z16, None)

            def hvec(b, g, carry, shift=shift, prefix=prefix, p=p):
                slot = ((b >> shift) & 0xFF) * 16 + lanes
                if p == 0:
                    plsc.addupdate_scatter(hist16, [slot], ones)
                else:
                    m = (b >> (shift + 8)) == prefix
                    plsc.addupdate_scatter(hist16, [slot], ones, mask=m)
                return carry
            scan(hvec, None)

            def red(gq, _):
                def addl(l, acc):
                    return acc + hist16[pl.ds(l * 256 + gq * 16, 16)]
                histl[pl.ds(gq * 16, 16)] = lax.fori_loop(
                    0, 16, addl, jnp.zeros((16,), jnp.int32))
                return None
            lax.fori_loop(0, 16, red, None)

            pltpu.sync_copy(histl, cntall.at[pl.ds(w * 256, 256)])
            plsc.subcore_barrier()
            pltpu.sync_copy(cntall, hist16)
            plsc.subcore_barrier()
            def gred(gq, _):
                def addu(u, acc):
                    return acc + hist16[pl.ds(u * 256 + gq * 16, 16)]
                ghist[pl.ds(gq * 16, 16)] = lax.fori_loop(
                    0, 16, addu, jnp.zeros((16,), jnp.int32))
                return None
            lax.fori_loop(0, 16, gred, None)

            def find(gq, carry):
                running, bucket, c_less = carry
                v = ghist[pl.ds(gq * 16, 16)]
                cum = plsc.cumsum(v)
                tot = jnp.max(cum)
                ffs = jnp.max(plsc.all_reduce_ffs((running + cum) >= k_rem))
                less = jnp.sum(jnp.where(lanes < ffs, v, 0))
                new = jnp.logical_and(bucket < 0, ffs < 16)
                bucket = jnp.where(new, gq * 16 + ffs, bucket)
                c_less = jnp.where(new, running + less, c_less)
                return running + tot, bucket, c_less
            _, bucket, c_less = lax.fori_loop(
                0, 16, find, (jnp.int32(0), jnp.int32(-1), jnp.int32(0)))

            def lcnt(gq, carry):
                ll, le = carry
                v = histl[pl.ds(gq * 16, 16)]
                slot = gq * 16 + lanes
                ll = ll + jnp.sum(jnp.where(slot < bucket, v, 0))
                le = le + jnp.sum(jnp.where(slot == bucket, v, 0))
                return ll, le
            ll, le = lax.fori_loop(0, 16, lcnt, (jnp.int32(0), jnp.int32(0)))
            local_lt = local_lt + ll
            n_eq_w = le
            total_lt = total_lt + c_less
            k_rem = k_rem - c_less
            prefix = (prefix << 8) | bucket

        thresh = prefix

        cnt16[...] = jnp.where(lanes == 0, local_lt,
                               jnp.where(lanes == 1, n_eq_w, 0))
        pltpu.sync_copy(cnt16, cntall.at[pl.ds(w * 16, 16)])
        plsc.subcore_barrier()
        pltpu.sync_copy(cntall.at[pl.ds(0, 256)], hist16.at[pl.ds(0, 256)])
        lt_off = jnp.int32(0)
        eq_off = jnp.int32(0)
        for u in range(_NSUB):
            take = u < w
            cu = hist16[pl.ds(u * 16, 16)]
            lt_off = lt_off + jnp.where(take, cu[0], 0)
            eq_off = eq_off + jnp.where(take, cu[1], 0)

        def init_pos(i, _):
            pos2d[i >> 3, pl.ds((i & 7) * 16, 16)] = jnp.full(
                (16,), _KPAD - 2, jnp.int32)
            return None
        lax.fori_loop(0, _KPAD // 16, init_pos, None)

        def evec(b, g, carry):
            ltc, eqr, lc = carry
            lt = b < thresh
            eq = b == thresh
            pos_lt = ltc + plsc.cumsum(lt.astype(jnp.int32)) - 1
            reqr = eqr + plsc.cumsum(eq.astype(jnp.int32)) - 1
            acc = jnp.logical_and(eq, reqr < k_rem)
            pos = jnp.where(lt, pos_lt, total_lt + reqr)
            sel = jnp.logical_or(lt, acc)
            slot = lc + plsc.cumsum(sel.astype(jnp.int32)) - 1
            plsc.store_scatter(vals2d, [slot >> 7, slot & 127], g, mask=sel)
            plsc.store_scatter(pos2d, [slot >> 7, slot & 127], pos, mask=sel)
            nlt = jnp.max(plsc.all_reduce_population_count(lt))
            neq = jnp.max(plsc.all_reduce_population_count(eq))
            nsel = jnp.max(plsc.all_reduce_population_count(sel))
            return ltc + nlt, eqr + neq, lc + nsel
        _, _, n_sel_w = scan(evec, (lt_off, eq_off, jnp.int32(0)))

        nrows = (n_sel_w + 127) >> 7
        def put(j, _):
            pltpu.async_copy(vals2d.at[j], sel_hbm.at[pos2d.at[j]], sem)
            return None
        lax.fori_loop(0, nrows, put, None)
        def drain(j, _):
            pltpu.make_async_copy(vals2d.at[j], sel_hbm.at[pos2d.at[j]],
                                  sem).wait()
            return None
        lax.fori_loop(0, nrows, drain, None)

    sel_fn = pl.kernel(
        body,
        out_type=jax.ShapeDtypeStruct((_KPAD,), jnp.int32),
        mesh=mesh,
        compiler_params=pltpu.CompilerParams(needs_layout_passes=False),
        scratch_types=[
            pltpu.VMEM((_CH,), jnp.int32),
            pltpu.VMEM((4096,), jnp.int32),
            pltpu.VMEM((256,), jnp.int32),
            pltpu.VMEM((256,), jnp.int32),
            pltpu.VMEM((16,), jnp.int32),
            pltpu.VMEM_SHARED((4096,), jnp.int32),
            pltpu.VMEM((_KPAD // 128, 128), jnp.int32),
            pltpu.VMEM((_KPAD // 128, 128), jnp.int32),
            pltpu.SemaphoreType.DMA,
        ],
    )

    def run(flat):
        bits = lax.bitcast_convert_type(jnp.abs(flat - 0.5), jnp.int32)
        pad = jnp.full((n_pad - n,), 0x7F000000, jnp.int32)
        return sel_fn(jnp.concatenate([bits, pad]))[:k]

    return run


def kernel(feats, W1, b1, W2, b2, W3, b3):
    feats2 = feats.reshape(1, 256)
    w1p = jnp.zeros((128, 256), jnp.float32).at[:3].set(W1[:3])
    w1f = W1[3:]
    b1r = b1.reshape(1, 256)
    b2r = b2.reshape(1, 256)
    w3p = jnp.zeros((256, 128), jnp.float32).at[:, :1].set(W3)
    b3p = jnp.zeros((1, 128), jnp.float32).at[0, 0].set(b3[0])
    mlp = functools.partial(_mlp_eval, feats=feats2, w1p=w1p, w1f=w1f, b1=b1r,
                            w2=W2, b2=b2r, w3p=w3p, b3p=b3p)

    r0 = _RESOLUTIONS[0]
    a = jnp.linspace(0, _FINAL - 1, r0).astype(jnp.int32)
    gz, gy, gx = jnp.meshgrid(a, a, a, indexing='ij')
    coords0 = jnp.stack([gx, gy, gz], axis=0).reshape(3, -1).T
    occ0 = mlp(_pad_points(_scale(coords0)))[:r0 ** 3, 0]
    vol = occ0.reshape(r0, r0, r0)

    for res, num_pt in zip(_RESOLUTIONS[1:], _NUM_POINTS[1:]):
        stride = (_FINAL - 1) // (res - 1)
        vol = _upsample(vol)
        n = res ** 3
        flat = vol.reshape(n)
        idx = _make_select(n, num_pt)(flat)
        xi = idx % res
        yi = (idx // res) % res
        zi = idx // (res * res)
        coords = jnp.stack([xi, yi, zi], axis=-1) * stride
        vals = mlp(_pad_points(_scale(coords)))[:num_pt, 0]
        flat = flat.at[idx].set(vals)
        vol = flat.reshape(res, res, res)

    return vol.reshape(1, 1, _FINAL, _FINAL, _FINAL)
